# baseline (device time: 17867 ns/iter reference)
import jax
import jax.numpy as jnp
from jax import lax
from jax.experimental import pallas as pl
from jax.experimental.pallas import tpu as pltpu

N_DEV = 4
C = 2


def kernel(x):
    m, n = x.shape
    half = m // 2
    quar = m // 4
    nh = n // 2
    w = nh // C

    def body(x_ref, out_ref, buf1, buf2, send_sems, recv_sems):
        p = lax.axis_index("i")
        p1 = jnp.bitwise_xor(p, 1)
        p2 = 3 - p

        barrier_sem = pltpu.get_barrier_semaphore()
        for nbr in (p1, p2):
            pl.semaphore_signal(
                barrier_sem, inc=1,
                device_id=(nbr,), device_id_type=pl.DeviceIdType.MESH,
            )
        pl.semaphore_wait(barrier_sem, 2)

        keep_a = jnp.where((p == 0) | (p == 3), 0, half)
        keep_b = jnp.where(p < 2, 0, half)
        sec_a = jnp.where(p >= 2, quar, 0)
        sec_b = jnp.where((p == 1) | (p == 3), quar, 0)
        own_a, part_a = keep_a + sec_a, keep_a + quar - sec_a
        own_b, part_b = keep_b + sec_b, keep_b + quar - sec_b
        po_a, pp_a = half - keep_a + sec_a, half - keep_a + quar - sec_a
        po_b, pp_b = half - keep_b + quar - sec_b, half - keep_b + sec_b

        keeps = [keep_a, keep_b]
        owns = [own_a, own_b]
        parts = [part_a, part_b]
        peer_owns = [po_a, po_b]
        peer_parts = [pp_a, pp_b]
        orders = [(p1, p2, p2, p1), (p2, p1, p1, p2)]

        def col(d, c):
            return pl.ds(d * nh + c * w, w)

        def copy(src, dst, k, dev):
            return pltpu.make_async_remote_copy(
                src_ref=src,
                dst_ref=dst,
                send_sem=send_sems.at[k],
                recv_sem=recv_sems.at[k],
                device_id=(dev,),
                device_id_type=pl.DeviceIdType.MESH,
            )

        chains = [(d, c) for c in range(C) for d in range(2)]
        kidx = {(j, d, c): (d * C + c) * 6 + j
                for d, c in chains for j in range(6)}
        rd = {}

        for d, c in chains:
            sl = d * C + c
            rd[(0, d, c)] = copy(x_ref.at[pl.ds(peer_parts[d], quar), col(d, c)],
                                 buf1.at[sl, pl.ds(0, quar)],
                                 kidx[(0, d, c)], orders[d][0])
            rd[(0, d, c)].start()
        for d, c in chains:
            sl = d * C + c
            rd[(1, d, c)] = copy(x_ref.at[pl.ds(peer_owns[d], quar), col(d, c)],
                                 buf1.at[sl, pl.ds(quar, quar)],
                                 kidx[(1, d, c)], orders[d][0])
            rd[(1, d, c)].start()

        for d, c in chains:
            rd[(0, d, c)].wait_recv()
            sl, q = d * C + c, pl.ds(parts[d], quar)
            out_ref[q, col(d, c)] = x_ref[q, col(d, c)] + buf1[sl, 0:quar, :]
            rd[(2, d, c)] = copy(out_ref.at[q, col(d, c)], buf2.at[sl],
                                 kidx[(2, d, c)], orders[d][1])
            rd[(2, d, c)].start()

        for d, c in chains:
            rd[(1, d, c)].wait_recv()
            sl, q = d * C + c, pl.ds(owns[d], quar)
            out_ref[q, col(d, c)] = x_ref[q, col(d, c)] + buf1[sl, quar:half, :]

        for d, c in chains:
            rd[(2, d, c)].wait_recv()
            sl, q = d * C + c, pl.ds(owns[d], quar)
            out_ref[q, col(d, c)] = out_ref[q, col(d, c)] + buf2[sl, :, :]
            rd[(3, d, c)] = copy(out_ref.at[q, col(d, c)],
                                 out_ref.at[q, col(d, c)],
                                 kidx[(3, d, c)], orders[d][2])
            rd[(3, d, c)].start()
            rd[(4, d, c)] = copy(out_ref.at[q, col(d, c)],
                                 out_ref.at[q, col(d, c)],
                                 kidx[(4, d, c)], orders[d][3])
            rd[(4, d, c)].start()

        for d, c in chains:
            rd[(3, d, c)].wait_recv()
            q = pl.ds(parts[d], quar)
            rd[(5, d, c)] = copy(out_ref.at[q, col(d, c)],
                                 out_ref.at[q, col(d, c)],
                                 kidx[(5, d, c)], orders[d][3])
            rd[(5, d, c)].start()

        for d, c in chains:
            rd[(4, d, c)].wait_recv()
            rd[(5, d, c)].wait_recv()

        for r in rd.values():
            r.wait_send()

    return pl.pallas_call(
        body,
        out_shape=jax.ShapeDtypeStruct((m, n), x.dtype),
        in_specs=[pl.BlockSpec(memory_space=pltpu.VMEM)],
        out_specs=pl.BlockSpec(memory_space=pltpu.VMEM),
        scratch_shapes=[
            pltpu.VMEM((2 * C, half, w), x.dtype),
            pltpu.VMEM((2 * C, quar, w), x.dtype),
            pltpu.SemaphoreType.DMA((12 * C,)),
            pltpu.SemaphoreType.DMA((12 * C,)),
        ],
        compiler_params=pltpu.CompilerParams(collective_id=0),
    )(x)


# device time: 16286 ns/iter; 1.0971x vs baseline; 1.0971x over previous
import jax
import jax.numpy as jnp
from jax import lax
from jax.experimental import pallas as pl
from jax.experimental.pallas import tpu as pltpu

N_DEV = 4
C = 2


def kernel(x):
    m, n = x.shape
    half = m // 2
    nh = n // 2
    w = nh // C

    def body(x_ref, out_ref, buf1, buf2, send_sems, recv_sems):
        p = lax.axis_index("i")
        p1 = jnp.bitwise_xor(p, 1)
        p2 = 3 - p

        barrier_sem = pltpu.get_barrier_semaphore()
        for nbr in (p1, p2):
            pl.semaphore_signal(
                barrier_sem, inc=1,
                device_id=(nbr,), device_id_type=pl.DeviceIdType.MESH,
            )
        pl.semaphore_wait(barrier_sem, 2)

        keep_a = jnp.where((p == 0) | (p == 3), 0, half)
        keep_b = jnp.where(p < 2, 0, half)
        keeps = [keep_a, keep_b]
        orders = [(p1, p2, p1), (p2, p1, p2)]

        def col(d, c):
            return pl.ds(d * nh + c * w, w)

        def copy(src, dst, k, dev):
            return pltpu.make_async_remote_copy(
                src_ref=src,
                dst_ref=dst,
                send_sem=send_sems.at[k],
                recv_sem=recv_sems.at[k],
                device_id=(dev,),
                device_id_type=pl.DeviceIdType.MESH,
            )

        chains = [(d, c) for c in range(C) for d in range(2)]
        kidx = {(j, d, c): (d * C + c) * 3 + j
                for d, c in chains for j in range(3)}
        rd = {}

        for d, c in chains:
            sl = d * C + c
            r = copy(x_ref.at[pl.ds(half - keeps[d], half), col(d, c)],
                     buf1.at[sl], kidx[(0, d, c)], orders[d][0])
            rd[(0, d, c)] = r
            r.start()

        for d, c in chains:
            rd[(0, d, c)].wait_recv()
            sl, s = d * C + c, pl.ds(keeps[d], half)
            out_ref[s, col(d, c)] = x_ref[s, col(d, c)] + buf1[sl]
            r = copy(out_ref.at[s, col(d, c)], buf2.at[sl],
                     kidx[(1, d, c)], orders[d][1])
            rd[(1, d, c)] = r
            r.start()

        for d, c in chains:
            rd[(1, d, c)].wait_recv()
            sl, s = d * C + c, pl.ds(keeps[d], half)
            out_ref[s, col(d, c)] = out_ref[s, col(d, c)] + buf2[sl]
            r = copy(out_ref.at[s, col(d, c)], out_ref.at[s, col(d, c)],
                     kidx[(2, d, c)], orders[d][2])
            rd[(2, d, c)] = r
            r.start()

        for d, c in chains:
            rd[(2, d, c)].wait_recv()

        for r in rd.values():
            r.wait_send()

    return pl.pallas_call(
        body,
        out_shape=jax.ShapeDtypeStruct((m, n), x.dtype),
        in_specs=[pl.BlockSpec(memory_space=pltpu.VMEM)],
        out_specs=pl.BlockSpec(memory_space=pltpu.VMEM),
        scratch_shapes=[
            pltpu.VMEM((2 * C, half, w), x.dtype),
            pltpu.VMEM((2 * C, half, w), x.dtype),
            pltpu.SemaphoreType.DMA((6 * C,)),
            pltpu.SemaphoreType.DMA((6 * C,)),
        ],
        compiler_params=pltpu.CompilerParams(collective_id=0),
    )(x)
